# Initial kernel scaffold; baseline (speedup 1.0000x reference)
import jax
import jax.numpy as jnp
from jax.experimental import pallas as pl


def kernel(atom_fea, nbr_fea, nbr_fea_idx, crystal_atom_idx, params):
    raise NotImplementedError("write your pallas kernel here")


# speedup vs baseline: 2.4371x; 2.4371x over previous
import functools

import jax
import jax.numpy as jnp
from jax import lax
from jax.experimental import pallas as pl
from jax.experimental.pallas import tpu as pltpu
from jax.experimental.pallas import tpu_sc as plsc

_F32 = jnp.float32


def _softplus(v):
    return jnp.maximum(v, 0.0) + jnp.log1p(jnp.exp(-jnp.abs(v)))


def _sigmoid(v):
    return 1.0 / (1.0 + jnp.exp(-v))


def _sc_gather(table, idx2):
    n_chunks, C = idx2.shape
    F = table.shape[1]
    info = plsc.get_sparse_core_info()
    NC, NS = info.num_cores, info.num_subcores
    NW = NC * NS
    mesh = plsc.VectorSubcoreMesh(core_axis_name="c", subcore_axis_name="s")

    @functools.partial(
        pl.kernel,
        out_type=jax.ShapeDtypeStruct((n_chunks * C, F), _F32),
        mesh=mesh,
        scratch_types=[
            pltpu.VMEM((C,), jnp.int32),
            pltpu.VMEM((C, F), _F32),
            pltpu.SemaphoreType.DMA,
        ],
    )
    def gk(table_hbm, idx_hbm, out_hbm, idx_v, rows_v, sem):
        wid = lax.axis_index("s") * NC + lax.axis_index("c")
        base = n_chunks // NW
        extra = n_chunks - base * NW
        my_n = base + (wid < extra).astype(jnp.int32)
        my_start = wid * base + jnp.minimum(wid, extra)

        def body(k, carry):
            ch = my_start + k
            pltpu.sync_copy(idx_hbm.at[ch], idx_v)
            pltpu.async_copy(table_hbm.at[idx_v], rows_v, sem).wait()
            pltpu.sync_copy(rows_v, out_hbm.at[pl.ds(ch * C, C)])
            return carry

        lax.fori_loop(0, my_n, body, 0)

    return gk(table, idx2)


def _emb_body(af_ref, w_ref, b_ref, wn_ref, o_ref, z_ref):
    x = jnp.dot(af_ref[...], w_ref[...], preferred_element_type=_F32) + b_ref[...]
    o_ref[...] = x
    z_ref[...] = jnp.dot(x, wn_ref[...], preferred_element_type=_F32)


def _p1_body(M, x_ref, zg_ref, nb_ref, ws_ref, we_ref, b_ref,
             ssum_ref, ssq_ref):
    i = pl.program_id(0)
    s = jnp.dot(x_ref[...], ws_ref[...], preferred_element_type=_F32) + b_ref[...]
    acc1 = jnp.zeros_like(ssum_ref)
    acc2 = jnp.zeros_like(ssq_ref)
    for m in range(M):
        y = (
            s
            + zg_ref[m]
            + jnp.dot(nb_ref[m], we_ref[...], preferred_element_type=_F32)
        )
        acc1 += jnp.sum(y, axis=0, keepdims=True)
        acc2 += jnp.sum(y * y, axis=0, keepdims=True)

    @pl.when(i == 0)
    def _():
        ssum_ref[...] = jnp.zeros_like(ssum_ref)
        ssq_ref[...] = jnp.zeros_like(ssq_ref)

    ssum_ref[...] += acc1
    ssq_ref[...] += acc2


def _p2_body(M, F, cnt1, x_ref, zg_ref, nb_ref, ws_ref, we_ref, b_ref,
             ssum_ref, ssq_ref, g1_ref, b1_ref, t_ref, ts_ref, tq_ref):
    i = pl.program_id(0)
    inv = 1.0 / cnt1
    mean = ssum_ref[...] * inv
    var = ssq_ref[...] * inv - mean * mean
    scale = g1_ref[...] * lax.rsqrt(var + 1e-5)
    shift = b1_ref[...] - mean * scale
    s = jnp.dot(x_ref[...], ws_ref[...], preferred_element_type=_F32) + b_ref[...]
    t = jnp.zeros_like(t_ref)
    for m in range(M):
        y = (
            s
            + zg_ref[m]
            + jnp.dot(nb_ref[m], we_ref[...], preferred_element_type=_F32)
        )
        yh = y * scale + shift
        t += _sigmoid(yh[:, :F]) * _softplus(yh[:, F:])
    t_ref[...] = t

    @pl.when(i == 0)
    def _():
        ts_ref[...] = jnp.zeros_like(ts_ref)
        tq_ref[...] = jnp.zeros_like(tq_ref)

    ts_ref[...] += jnp.sum(t, axis=0, keepdims=True)
    tq_ref[...] += jnp.sum(t * t, axis=0, keepdims=True)


def _p3_body(cnt2, x_ref, t_ref, ts_ref, tq_ref, g2_ref, b2_ref, xo_ref):
    inv = 1.0 / cnt2
    mean = ts_ref[...] * inv
    var = tq_ref[...] * inv - mean * mean
    scale = g2_ref[...] * lax.rsqrt(var + 1e-5)
    shift = b2_ref[...] - mean * scale
    xo_ref[...] = _softplus(x_ref[...] + t_ref[...] * scale + shift)


def _p3z_body(cnt2, x_ref, t_ref, ts_ref, tq_ref, g2_ref, b2_ref, wn_ref,
              xo_ref, z_ref):
    inv = 1.0 / cnt2
    mean = ts_ref[...] * inv
    var = tq_ref[...] * inv - mean * mean
    scale = g2_ref[...] * lax.rsqrt(var + 1e-5)
    shift = b2_ref[...] - mean * scale
    xn = _softplus(x_ref[...] + t_ref[...] * scale + shift)
    xo_ref[...] = xn
    z_ref[...] = jnp.dot(xn, wn_ref[...], preferred_element_type=_F32)


def _pool_body(x_ref, p_ref, wf_ref, bf_ref, wo_ref, bo_ref, o_ref):
    crys = jnp.dot(p_ref[...], x_ref[...], preferred_element_type=_F32)
    crys = _softplus(crys)
    h = _softplus(
        jnp.dot(crys, wf_ref[...], preferred_element_type=_F32) + bf_ref[...]
    )
    o_ref[...] = jnp.dot(h, wo_ref[...], preferred_element_type=_F32) + bo_ref[...]


def _const_spec(shape):
    return pl.BlockSpec(shape, lambda i: tuple(0 for _ in shape))


def kernel(atom_fea, nbr_fea, nbr_fea_idx, crystal_atom_idx, params):
    N, ORIG = atom_fea.shape
    _, M, NBR = nbr_fea.shape
    F = params["embedding"]["W"].shape[1]
    H = params["conv_to_fc"]["W"].shape[1]
    N0, A = crystal_atom_idx.shape

    Bn = 1000
    grid = (N // Bn,)
    arb = pltpu.CompilerParams(dimension_semantics=("arbitrary",))

    idx_flat = jnp.transpose(nbr_fea_idx).astype(jnp.int32).reshape(-1)
    C = 128
    idx2 = idx_flat.reshape((M * N) // C, C)
    nbrT = jnp.transpose(nbr_fea, (1, 0, 2))

    emb_w = params["embedding"]["W"]
    emb_b = params["embedding"]["b"].reshape(1, F)

    convs = params["convs"]
    NL = len(convs)
    wn_of = [c["fc_full"]["W"][F : 2 * F] for c in convs]

    x, z = pl.pallas_call(
        _emb_body,
        grid=grid,
        in_specs=[
            pl.BlockSpec((Bn, ORIG), lambda i: (i, 0)),
            _const_spec((ORIG, F)),
            _const_spec((1, F)),
            _const_spec((F, 2 * F)),
        ],
        out_specs=[
            pl.BlockSpec((Bn, F), lambda i: (i, 0)),
            pl.BlockSpec((Bn, 2 * F), lambda i: (i, 0)),
        ],
        out_shape=[
            jax.ShapeDtypeStruct((N, F), _F32),
            jax.ShapeDtypeStruct((N, 2 * F), _F32),
        ],
        compiler_params=arb,
    )(atom_fea, emb_w, emb_b, wn_of[0])

    for li, c in enumerate(convs):
        wfull = c["fc_full"]["W"]
        ws = wfull[:F]
        we = wfull[2 * F :]
        bfull = c["fc_full"]["b"].reshape(1, 2 * F)
        g1 = c["bn1_g"].reshape(1, 2 * F)
        b1 = c["bn1_b"].reshape(1, 2 * F)
        g2 = c["bn2_g"].reshape(1, F)
        b2 = c["bn2_b"].reshape(1, F)

        zg3 = _sc_gather(z, idx2).reshape(M, N, 2 * F)

        edge_specs = [
            pl.BlockSpec((Bn, F), lambda i: (i, 0)),
            pl.BlockSpec((M, Bn, 2 * F), lambda i: (0, i, 0)),
            pl.BlockSpec((M, Bn, NBR), lambda i: (0, i, 0)),
            _const_spec((F, 2 * F)),
            _const_spec((NBR, 2 * F)),
            _const_spec((1, 2 * F)),
        ]

        ssum, ssq = pl.pallas_call(
            functools.partial(_p1_body, M),
            grid=grid,
            in_specs=edge_specs,
            out_specs=[_const_spec((1, 2 * F))] * 2,
            out_shape=[jax.ShapeDtypeStruct((1, 2 * F), _F32)] * 2,
            compiler_params=arb,
        )(x, zg3, nbrT, ws, we, bfull)

        t, ts, tq = pl.pallas_call(
            functools.partial(_p2_body, M, F, float(N * M)),
            grid=grid,
            in_specs=edge_specs
            + [_const_spec((1, 2 * F))] * 2
            + [_const_spec((1, 2 * F))] * 2,
            out_specs=[
                pl.BlockSpec((Bn, F), lambda i: (i, 0)),
                _const_spec((1, F)),
                _const_spec((1, F)),
            ],
            out_shape=[
                jax.ShapeDtypeStruct((N, F), _F32),
                jax.ShapeDtypeStruct((1, F), _F32),
                jax.ShapeDtypeStruct((1, F), _F32),
            ],
            compiler_params=arb,
        )(x, zg3, nbrT, ws, we, bfull, ssum, ssq, g1, b1)

        p3_specs = [
            pl.BlockSpec((Bn, F), lambda i: (i, 0)),
            pl.BlockSpec((Bn, F), lambda i: (i, 0)),
            _const_spec((1, F)),
            _const_spec((1, F)),
            _const_spec((1, F)),
            _const_spec((1, F)),
        ]
        if li + 1 < NL:
            x, z = pl.pallas_call(
                functools.partial(_p3z_body, float(N)),
                grid=grid,
                in_specs=p3_specs + [_const_spec((F, 2 * F))],
                out_specs=[
                    pl.BlockSpec((Bn, F), lambda i: (i, 0)),
                    pl.BlockSpec((Bn, 2 * F), lambda i: (i, 0)),
                ],
                out_shape=[
                    jax.ShapeDtypeStruct((N, F), _F32),
                    jax.ShapeDtypeStruct((N, 2 * F), _F32),
                ],
                compiler_params=arb,
            )(x, t, ts, tq, g2, b2, wn_of[li + 1])
        else:
            x = pl.pallas_call(
                functools.partial(_p3_body, float(N)),
                grid=grid,
                in_specs=p3_specs,
                out_specs=pl.BlockSpec((Bn, F), lambda i: (i, 0)),
                out_shape=jax.ShapeDtypeStruct((N, F), _F32),
                compiler_params=arb,
            )(x, t, ts, tq, g2, b2)

    BC = 64
    N0p = ((N0 + BC - 1) // BC) * BC
    BA = BC * A
    x_pad = jnp.concatenate([x, jnp.zeros(((N0p - N0) * A, F), _F32)], axis=0)
    pool_p = jnp.kron(jnp.eye(BC, dtype=_F32), jnp.full((1, A), 1.0 / A, _F32))
    wf = params["conv_to_fc"]["W"]
    bf = params["conv_to_fc"]["b"].reshape(1, H)
    wo = params["fc_out"]["W"]
    bo = params["fc_out"]["b"].reshape(1, 1)

    out = pl.pallas_call(
        _pool_body,
        grid=(N0p // BC,),
        in_specs=[
            pl.BlockSpec((BA, F), lambda i: (i, 0)),
            _const_spec((BC, BA)),
            _const_spec((F, H)),
            _const_spec((1, H)),
            _const_spec((H, 1)),
            _const_spec((1, 1)),
        ],
        out_specs=pl.BlockSpec((BC, 1), lambda i: (i, 0)),
        out_shape=jax.ShapeDtypeStruct((N0p, 1), _F32),
        compiler_params=arb,
    )(x_pad, pool_p, wf, bf, wo, bo)
    return out[:N0]
